# Initial kernel scaffold; baseline (speedup 1.0000x reference)
import jax
import jax.numpy as jnp
from jax.experimental import pallas as pl


def kernel(x, edge_attr, params, edge_index, batch):
    raise NotImplementedError("write your pallas kernel here")


# speedup vs baseline: 31.0483x; 31.0483x over previous
import functools

import jax
import jax.numpy as jnp
from jax import lax
from jax.experimental import pallas as pl
from jax.experimental.pallas import tpu as pltpu
from jax.experimental.pallas import tpu_sc as plsc

N = 10000
E = 320000
D = 128
G = 64
LAYER_CFG = [(8, 16, True), (8, 16, True), (8, 16, True), (1, 128, False)]

NEG = -1e30
AC = 136

NC, NS = 2, 16
NT = NC * NS
CH_PRE = 79
EPRE_PAD = CH_PRE * 128 * NT
CH_MAIN = 81
EEXT_PAD = CH_MAIN * 128 * NT
NP = 10112
NPT = 632

COLB = [0, 8, 16, 24]
ONECOL = 33


def _aepre_body(ea_ref, w_ref, out_ref):
    res = jnp.dot(ea_ref[...], w_ref[...], preferred_element_type=jnp.float32)
    col = lax.broadcasted_iota(jnp.int32, res.shape, 1)
    out_ref[...] = jnp.where(col == ONECOL, 1.0, res)


def _aepre(edge_attr, wall40):
    br = 2000
    return pl.pallas_call(
        _aepre_body,
        grid=(E // br,),
        in_specs=[pl.BlockSpec((br, D), lambda i: (i, 0)),
                  pl.BlockSpec((D, 40), lambda i: (0, 0))],
        out_specs=pl.BlockSpec((br, 40), lambda i: (i, 0)),
        out_shape=jax.ShapeDtypeStruct((E, 40), jnp.float32),
    )(edge_attr, wall40)


def _aeself_body(acc_ref, out_ref):
    v = acc_ref[...]
    a = v[:N] + v[NP:NP + N]
    cnt = jnp.maximum(a[:, ONECOL:ONECOL + 1], 1.0)
    out_ref[...] = a / cnt


def _aeself(accpre):
    return pl.pallas_call(
        _aeself_body,
        out_shape=jax.ShapeDtypeStruct((N, 40), jnp.float32),
    )(accpre)


def _prep(h, lin_wT, ps, pd, pads):
    xl = jnp.dot(h, lin_wT, preferred_element_type=jnp.float32)
    asrc = jnp.dot(xl, ps, preferred_element_type=jnp.float32) + pads[0:1, :]
    adst = jnp.dot(xl, pd, preferred_element_type=jnp.float32) + pads[1:2, :]
    return xl, asrc, adst


def _prep0_body(x_ref, lw_ref, ps_ref, pd_ref, pads_ref,
                xl_ref, as_ref, ad_ref):
    xl, asrc, adst = _prep(x_ref[...], lw_ref[...], ps_ref[...], pd_ref[...],
                           pads_ref[...])
    xl_ref[...] = xl
    as_ref[...] = asrc
    ad_ref[...] = adst


def _prep0(x, lin_wT, ps, pd, pads):
    return pl.pallas_call(
        _prep0_body,
        out_shape=[jax.ShapeDtypeStruct((N, D), jnp.float32),
                   jax.ShapeDtypeStruct((N, 16), jnp.float32),
                   jax.ShapeDtypeStruct((N, 16), jnp.float32)],
    )(x, lin_wT, ps, pd, pads)


def _post_common(acc, hcur, heads, bias, bng, bnb, lng, lnb, skip_wT, skip_b):
    num = acc[:N, :D] + acc[NP:NP + N, :D]
    exs = acc[:N, D:AC] + acc[NP:NP + N, D:AC]
    if heads == 8:
        row = lax.broadcasted_iota(jnp.int32, (8, D), 0)
        col = lax.broadcasted_iota(jnp.int32, (8, D), 1)
        rep = jnp.where(col // 16 == row, 1.0, 0.0)
        denom = jnp.dot(exs, rep, preferred_element_type=jnp.float32)
    else:
        denom = exs[:, 0:1]
    hgat = num / (denom + 1e-16) + bias
    mu = jnp.mean(hgat, axis=0, keepdims=True)
    var = jnp.mean((hgat - mu) ** 2, axis=0, keepdims=True)
    hbn = (hgat - mu) / jnp.sqrt(var + 1e-5) * bng + bnb
    mu2 = jnp.mean(hbn, axis=1, keepdims=True)
    var2 = jnp.mean((hbn - mu2) ** 2, axis=1, keepdims=True)
    hln = (hbn - mu2) / jnp.sqrt(var2 + 1e-5) * lng + lnb
    skip = jnp.dot(hcur, skip_wT, preferred_element_type=jnp.float32) + skip_b
    z = hln + skip
    return jnp.where(z > 0, z, jnp.exp(jnp.minimum(z, 0.0)) - 1.0)


def _post_mid_body(acc_ref, h_ref, bias_ref, bng_ref, bnb_ref, lng_ref,
                   lnb_ref, sw_ref, sb_ref, lw_ref, ps_ref, pd_ref, pads_ref,
                   h_out, xl_out, as_out, ad_out, *, heads):
    hn = _post_common(acc_ref[...], h_ref[...], heads, bias_ref[...],
                      bng_ref[...], bnb_ref[...], lng_ref[...], lnb_ref[...],
                      sw_ref[...], sb_ref[...])
    h_out[...] = hn
    xl, asrc, adst = _prep(hn, lw_ref[...], ps_ref[...], pd_ref[...],
                           pads_ref[...])
    xl_out[...] = xl
    as_out[...] = asrc
    ad_out[...] = adst


def _post_mid(acc, h, heads, bias, bng, bnb, lng, lnb, sw, sb,
              lw, ps, pd, pads):
    return pl.pallas_call(
        functools.partial(_post_mid_body, heads=heads),
        out_shape=[jax.ShapeDtypeStruct((N, D), jnp.float32),
                   jax.ShapeDtypeStruct((N, D), jnp.float32),
                   jax.ShapeDtypeStruct((N, 16), jnp.float32),
                   jax.ShapeDtypeStruct((N, 16), jnp.float32)],
    )(acc, h, bias, bng, bnb, lng, lnb, sw, sb, lw, ps, pd, pads)


def _post_last_body(acc_ref, h_ref, bias_ref, bng_ref, bnb_ref, lng_ref,
                    lnb_ref, sw_ref, sb_ref, h_out, *, heads):
    h_out[...] = _post_common(acc_ref[...], h_ref[...], heads, bias_ref[...],
                              bng_ref[...], bnb_ref[...], lng_ref[...],
                              lnb_ref[...], sw_ref[...], sb_ref[...])


def _post_last(acc, h, heads, bias, bng, bnb, lng, lnb, sw, sb):
    return pl.pallas_call(
        functools.partial(_post_last_body, heads=heads),
        out_shape=jax.ShapeDtypeStruct((N, D), jnp.float32),
    )(acc, h, bias, bng, bnb, lng, lnb, sw, sb)


def _set2set_body(h_ref, batch_ref, wih_ref, whh_ref, bih_ref, bhh_ref,
                  pw_ref, pb_ref, out_ref):
    xs = h_ref[...]
    bb = batch_ref[...]
    gcol = lax.broadcasted_iota(jnp.int32, (N, G), 1)
    seg = jnp.where(bb == gcol, 1.0, 0.0)
    hh = jnp.zeros((G, D), jnp.float32)
    c = jnp.zeros((G, D), jnp.float32)
    q_star = jnp.zeros((G, 2 * D), jnp.float32)
    dn = (((0,), (0,)), ((), ()))
    for _ in range(3):
        gates = (jnp.dot(q_star, wih_ref[...], preferred_element_type=jnp.float32)
                 + bih_ref[...]
                 + jnp.dot(hh, whh_ref[...], preferred_element_type=jnp.float32)
                 + bhh_ref[...])
        gi = gates[:, 0:D]
        gf = gates[:, D:2 * D]
        gg = gates[:, 2 * D:3 * D]
        go = gates[:, 3 * D:4 * D]
        c = jax.nn.sigmoid(gf) * c + jax.nn.sigmoid(gi) * jnp.tanh(gg)
        hh = jax.nn.sigmoid(go) * jnp.tanh(c)
        e = jnp.sum(xs * jnp.dot(seg, hh, preferred_element_type=jnp.float32),
                    axis=1, keepdims=True)
        m = jnp.max(jnp.where(seg > 0, e, -jnp.inf), axis=0, keepdims=True)
        m = jnp.where(m > -1e37, m, 0.0)
        mn = jnp.dot(seg, m.T, preferred_element_type=jnp.float32)
        exn = jnp.exp(e - mn)
        s = lax.dot_general(seg, exn, dn,
                            preferred_element_type=jnp.float32)
        a = exn / (jnp.dot(seg, s, preferred_element_type=jnp.float32) + 1e-16)
        r = lax.dot_general(seg, a * xs, dn,
                            preferred_element_type=jnp.float32)
        q_star = jnp.concatenate([hh, r], axis=1)
    out = jnp.dot(q_star, pw_ref[...], preferred_element_type=jnp.float32) \
        + pb_ref[...]
    out_ref[...] = jnp.maximum(out, 0.0)


def _set2set(h, batch2d, wihT, whhT, bih, bhh, projT, projb):
    return pl.pallas_call(
        _set2set_body,
        out_shape=jax.ShapeDtypeStruct((G, D), jnp.float32),
    )(h, batch2d, wihT, whhT, bih, bhh, projT, projb)


def _sc_mesh():
    return plsc.VectorSubcoreMesh(core_axis_name="c", subcore_axis_name="s")


def _sc_pre_body(ae_hbm, dst_hbm, out_hbm, dsti, rowbuf, zbuf, acc, sem):
    c = lax.axis_index("c")
    s = lax.axis_index("s")
    tid = c * NS + s

    def zrow(i, _):
        zbuf[i, pl.ds(0, 16)] = jnp.zeros((16,), jnp.float32)
        zbuf[i, pl.ds(16, 16)] = jnp.zeros((16,), jnp.float32)
        zbuf[i, pl.ds(24, 16)] = jnp.zeros((16,), jnp.float32)
        return _
    lax.fori_loop(0, 104, zrow, None)
    for b in range(6):
        pltpu.sync_copy(zbuf, acc.at[pl.ds(s * NPT + b * 104, 104)])
    pltpu.sync_copy(zbuf.at[pl.ds(0, 8)], acc.at[pl.ds(s * NPT + 624, 8)])
    plsc.subcore_barrier()

    pltpu.sync_copy(dst_hbm.at[tid], dsti)

    def chunk(j, _):
        r0 = (tid * CH_PRE + j) * 128
        pltpu.async_copy(ae_hbm.at[pl.ds(r0, 128)], rowbuf, sem).wait()
        pltpu.sync_copy(rowbuf, acc.at[dsti.at[j]], add=True)
        return _
    lax.fori_loop(0, CH_PRE, chunk, None)
    plsc.subcore_barrier()
    pltpu.sync_copy(acc.at[pl.ds(s * NPT, NPT)],
                    out_hbm.at[pl.ds(c * NP + s * NPT, NPT)])


def _sc_pre(aepre_pad, dstp):
    k = pl.kernel(
        _sc_pre_body,
        out_type=jax.ShapeDtypeStruct((2 * NP, 40), jnp.float32),
        mesh=_sc_mesh(),
        compiler_params=pltpu.CompilerParams(use_tc_tiling_on_sc=False),
        scratch_types=[
            pltpu.VMEM((CH_PRE, 128), jnp.int32),
            pltpu.VMEM((128, 40), jnp.float32),
            pltpu.VMEM((104, 40), jnp.float32),
            pltpu.VMEM_SHARED((NP, 40), jnp.float32),
            pltpu.SemaphoreType.DMA,
        ],
    )
    return k(aepre_pad, dstp)


def _sc_edge_body(src_hbm, dst_hbm, ae_hbm, xl_hbm, as_hbm, ad_hbm, out_hbm,
                  srciv, dstiv, aebuf, asbuf, adbuf, xlbuf, msgbuf, acc,
                  sem, *, colb, heads, oc):
    c = lax.axis_index("c")
    s = lax.axis_index("s")
    tid = c * NS + s

    def zrow(i, _):
        for k9 in range(8):
            msgbuf[i, pl.ds(k9 * 16, 16)] = jnp.zeros((16,), jnp.float32)
        msgbuf[i, pl.ds(120, 16)] = jnp.zeros((16,), jnp.float32)
        return _
    lax.fori_loop(0, 128, zrow, None)
    for b in range(4):
        pltpu.sync_copy(msgbuf, acc.at[pl.ds(s * NPT + b * 128, 128)])
    pltpu.sync_copy(msgbuf.at[pl.ds(0, 120)],
                    acc.at[pl.ds(s * NPT + 512, 120)])
    plsc.subcore_barrier()

    def chunk(j, _):
        pltpu.sync_copy(src_hbm.at[tid, j], srciv)
        pltpu.sync_copy(dst_hbm.at[tid, j], dstiv)
        r0 = (tid * CH_MAIN + j) * 128
        cp_ae = pltpu.async_copy(ae_hbm.at[pl.ds(r0, 128)], aebuf, sem)
        cp_as = pltpu.async_copy(as_hbm.at[srciv], asbuf, sem)
        cp_ad = pltpu.async_copy(ad_hbm.at[dstiv], adbuf, sem)
        cp_xl = pltpu.async_copy(xl_hbm.at[srciv], xlbuf, sem)
        cp_ae.wait()
        cp_as.wait()
        cp_ad.wait()
        cp_xl.wait()

        def edge(e, _):
            av = asbuf[e, :] + adbuf[e, :] + aebuf[e, pl.ds(colb, 16)]
            av = jnp.where(av > 0, av, av * 0.2)
            ex = jnp.exp(av)
            msgbuf[e, pl.ds(120, 16)] = ex
            for h in range(heads):
                exs = ex[8 + h]
                for v in range(oc // 16):
                    col = h * oc + v * 16
                    msgbuf[e, pl.ds(col, 16)] = xlbuf[e, pl.ds(col, 16)] * exs
            return _
        lax.fori_loop(0, 128, edge, None)
        pltpu.sync_copy(msgbuf, acc.at[dstiv], add=True)
        return _
    lax.fori_loop(0, CH_MAIN, chunk, None)
    plsc.subcore_barrier()
    pltpu.sync_copy(acc.at[pl.ds(s * NPT, NPT)],
                    out_hbm.at[pl.ds(c * NP + s * NPT, NPT)])


def _sc_edge(srcx, dstx, aeall, xl, asrc, adst, colb, heads, oc):
    body = functools.partial(_sc_edge_body, colb=colb, heads=heads, oc=oc)
    k = pl.kernel(
        body,
        out_type=jax.ShapeDtypeStruct((2 * NP, AC), jnp.float32),
        mesh=_sc_mesh(),
        compiler_params=pltpu.CompilerParams(use_tc_tiling_on_sc=False),
        scratch_types=[
            pltpu.VMEM((128,), jnp.int32),
            pltpu.VMEM((128,), jnp.int32),
            pltpu.VMEM((128, 40), jnp.float32),
            pltpu.VMEM((128, 16), jnp.float32),
            pltpu.VMEM((128, 16), jnp.float32),
            pltpu.VMEM((128, D), jnp.float32),
            pltpu.VMEM((128, AC), jnp.float32),
            pltpu.VMEM_SHARED((NP, AC), jnp.float32),
            pltpu.SemaphoreType.DMA,
        ],
    )
    return k(srcx, dstx, aeall, xl, asrc, adst)


def _weight_prep(params):
    convs = params['convs']
    cols = []
    pmats = []
    for l, (h, o, _) in enumerate(LAYER_CFG):
        p = convs[l]
        lw = p['lin_edge_w'].reshape(h, o, D)
        cols.append(jnp.einsum('hod,ho->dh', lw, p['att_edge']))
        ps = jnp.zeros((D, 16), jnp.float32)
        pd = jnp.zeros((D, 16), jnp.float32)
        for hh in range(h):
            ps = ps.at[:, 8 + hh].set(
                jnp.zeros((D,)).at[hh * o:(hh + 1) * o].set(p['att_src'][hh]))
            pd = pd.at[:, 8 + hh].set(
                jnp.zeros((D,)).at[hh * o:(hh + 1) * o].set(p['att_dst'][hh]))
        padv = jnp.full((2, 16), NEG, jnp.float32)
        padv = padv.at[:, 8:8 + h].set(0.0)
        pmats.append((p['lin_w'].T, ps, pd, padv))
    wall = jnp.concatenate(cols, axis=1)
    wall40 = jnp.pad(wall, ((0, 0), (8, 7)))
    return wall40, pmats


def kernel(x, edge_attr, params, edge_index, batch):
    src = edge_index[0]
    dst = edge_index[1]
    wall40, pmats = _weight_prep(params)

    aepre = _aepre(edge_attr, wall40)

    aepre_pad = jnp.pad(aepre, ((0, EPRE_PAD - E), (0, 0)))
    dstp = jnp.pad(dst, (0, EPRE_PAD - E)).reshape(NT, CH_PRE, 128)
    accpre = _sc_pre(aepre_pad, dstp)
    aeself = _aeself(accpre)

    loop = jnp.arange(N, dtype=src.dtype)
    srcx = jnp.pad(jnp.concatenate([src, loop]), (0, EEXT_PAD - E - N))
    dstx = jnp.pad(jnp.concatenate([dst, loop]), (0, EEXT_PAD - E - N))
    srcx = srcx.reshape(NT, CH_MAIN, 128)
    dstx = dstx.reshape(NT, CH_MAIN, 128)
    aeall = jnp.concatenate([aepre, aeself], axis=0)
    aeall = jnp.pad(aeall, ((0, EEXT_PAD - E - N), (0, 0)),
                    constant_values=NEG)

    h = x
    lwT, ps, pd, padv = pmats[0]
    xl, asrc, adst = _prep0(x, lwT, ps, pd, padv)
    for l, (hd, oc, cc) in enumerate(LAYER_CFG):
        acc = _sc_edge(srcx, dstx, aeall, xl, asrc, adst,
                       COLB[l], hd, oc)
        p = params['convs'][l]
        bias = p['bias'].reshape(1, D)
        bng = params['bns'][l]['gamma'].reshape(1, D)
        bnb = params['bns'][l]['beta'].reshape(1, D)
        lng = params['lns'][l]['gamma'].reshape(1, D)
        lnb = params['lns'][l]['beta'].reshape(1, D)
        sw = params['skips'][l]['w'].T
        sb = params['skips'][l]['b'].reshape(1, D)
        if l < 3:
            lwT, ps, pd, padv = pmats[l + 1]
            h, xl, asrc, adst = _post_mid(acc, h, hd, bias, bng, bnb, lng,
                                          lnb, sw, sb, lwT, ps, pd, padv)
        else:
            h = _post_last(acc, h, hd, bias, bng, bnb, lng, lnb, sw, sb)

    lstm = params['lstm']
    proj = params['proj']
    return _set2set(h, batch.reshape(N, 1).astype(jnp.int32),
                    lstm['w_ih'].T, lstm['w_hh'].T,
                    lstm['b_ih'].reshape(1, 4 * D), lstm['b_hh'].reshape(1, 4 * D),
                    proj['w'].T, proj['b'].reshape(1, D))
